# baseline (speedup 1.0000x reference)
import functools

import jax
import jax.numpy as jnp
from jax import lax
from jax.experimental import pallas as pl
from jax.experimental.pallas import tpu as pltpu
from jax.experimental.pallas import tpu_sc as plsc

C, H, W = 9, 480, 640
B = 4
NC, NS = 2, 16
RS = 512
KE = 16384
G = H * W
GS = G // NS


def _round_bf16(h):
    bits = jax.lax.bitcast_convert_type(h, jnp.int32)
    r = bits + 0x7FFF + ((bits >> 16) & 1)
    r = r & jnp.int32(-65536)
    return jax.lax.bitcast_convert_type(r, jnp.float32)


def _tc_body(w1_ref, b1_ref, w2_ref, b2_ref, w3_ref, b3_ref,
             x_ref, y_ref, ts_ref, val_ref, idx_ref):
    t_ = ts_ref[0]
    idx_ref[0] = (x_ref[0] + float(W) * y_ref[0]).astype(jnp.int32)
    for c in range(C):
        s = t_ - float(c)
        h1 = []
        for j in range(16):
            z = w1_ref[j] * s + b1_ref[j]
            h = jnp.maximum(z, 0.1 * z)
            h1.append(_round_bf16(h))
        acc = None
        for i in range(16):
            z = b2_ref[i]
            for j in range(16):
                z = z + w2_ref[i, j] * h1[j]
            h2 = jnp.maximum(z, 0.1 * z)
            h2 = _round_bf16(h2)
            term = w3_ref[i] * h2
            acc = term if acc is None else acc + term
        val_ref[c, 0] = t_ * (acc + b3_ref[0])


def _tc_values(xq, yq, tsq, w1, b1, w2, b2, w3, b3):
    nb = xq.shape[1] // RS
    grid = (B, nb)
    ev_spec = pl.BlockSpec((1, RS, 128), lambda b_, r: (b_, r, 0))
    smem = pl.BlockSpec(memory_space=pltpu.SMEM)
    return pl.pallas_call(
        _tc_body,
        grid=grid,
        in_specs=[smem] * 6 + [ev_spec] * 3,
        out_specs=[
            pl.BlockSpec((C, 1, RS, 128), lambda b_, r: (0, b_, r, 0)),
            ev_spec,
        ],
        out_shape=[
            jax.ShapeDtypeStruct((C, B, xq.shape[1], 128), jnp.float32),
            jax.ShapeDtypeStruct((B, xq.shape[1], 128), jnp.int32),
        ],
        compiler_params=pltpu.CompilerParams(
            dimension_semantics=("parallel", "parallel")),
    )(w1, b1, w2, b2, w3, b3, xq, yq, tsq)


def _sc_scatter(values, idx, np_):
    ev_per_tile = np_ // NS
    n_chunks = ev_per_tile // KE
    b_per_core = B // NC
    mesh = plsc.VectorSubcoreMesh(core_axis_name="c", subcore_axis_name="s")

    @functools.partial(
        pl.kernel,
        out_type=jax.ShapeDtypeStruct((B * C * G,), jnp.float32),
        mesh=mesh,
        scratch_types=[
            pltpu.VMEM((KE,), jnp.int32),
            pltpu.VMEM((KE,), jnp.float32),
            pltpu.VMEM((GS,), jnp.float32),
            pltpu.VMEM_SHARED((G,), jnp.float32),
        ],
    )
    def run(val_hbm, idx_hbm, out_hbm, idx_v, val_v, zero_v, grid):
        cid = lax.axis_index("c")
        sid = lax.axis_index("s")

        def zbody(i, _):
            zero_v[pl.ds(i * 16, 16)] = jnp.zeros((16,), jnp.float32)
            return 0

        lax.fori_loop(0, GS // 16, zbody, 0)
        for bb in range(b_per_core):
            b = cid * b_per_core + bb
            for c in range(C):
                pltpu.sync_copy(zero_v, grid.at[pl.ds(sid * GS, GS)])
                plsc.subcore_barrier()

                def chunk(k, _):
                    off = sid * ev_per_tile + k * KE
                    pltpu.sync_copy(idx_hbm.at[pl.ds(b * np_ + off, KE)], idx_v)
                    pltpu.sync_copy(
                        val_hbm.at[pl.ds((c * B + b) * np_ + off, KE)], val_v)
                    pltpu.sync_copy(val_v, grid.at[idx_v], add=True)
                    return 0

                lax.fori_loop(0, n_chunks, chunk, 0)
                plsc.subcore_barrier()
                pltpu.sync_copy(grid.at[pl.ds(sid * GS, GS)],
                                out_hbm.at[pl.ds((b * C + c) * G + sid * GS, GS)])

    return run(values, idx)


def kernel(events, W1, b1, W2, b2, W3, b3):
    n = events.shape[0]
    n_per = n // B
    quantum = NS * KE
    np_ = ((n_per + quantum - 1) // quantum) * quantum
    pad = np_ - n_per

    cols = events.reshape(B, n_per, 5)
    x = cols[:, :, 0]
    y = cols[:, :, 1]
    p = cols[:, :, 3]
    t = events[:, 2]
    bcol = events[:, 4]
    for bi in range(B):
        mask = bcol == bi
        first_idx = jnp.argmax(mask)
        last_idx = n - 1 - jnp.argmax(mask[::-1])
        tb0 = t[first_idx]
        tb1 = t[last_idx]
        dtb = tb1 - tb0
        t_norm = (t - tb0) / jnp.where(dtb == 0, 1.0, dtb) * (C - 1)
        t = jnp.where(mask & (dtb != 0), t_norm, t)
    ts = (events[:, 3] * t).reshape(B, n_per)

    def prep(col):
        return jnp.pad(col, ((0, 0), (0, pad))).reshape(B, np_ // 128, 128)

    w2q = _round_bf16(W2)
    w3q = _round_bf16(W3)
    values, idx = _tc_values(prep(x), prep(y), prep(ts),
                             W1[:, 0], b1, w2q, b2, w3q[0], b3)
    out = _sc_scatter(values.reshape(C * B * np_), idx.reshape(B * np_), np_)
    return out.reshape(B, C, H, W)


# speedup vs baseline: 1.0535x; 1.0535x over previous
import functools

import jax
import jax.numpy as jnp
from jax import lax
from jax.experimental import pallas as pl
from jax.experimental.pallas import tpu as pltpu
from jax.experimental.pallas import tpu_sc as plsc

C, H, W = 9, 480, 640
B = 4
NC, NS = 2, 16
RS = 512
KE = 16384
G = H * W
GS = G // NS


def _round_bf16(h):
    bits = jax.lax.bitcast_convert_type(h, jnp.int32)
    r = bits + 0x7FFF + ((bits >> 16) & 1)
    r = r & jnp.int32(-65536)
    return jax.lax.bitcast_convert_type(r, jnp.float32)


def _tc_body(w1_ref, b1_ref, w2_ref, b2_ref, w3_ref, b3_ref,
             x_ref, y_ref, ts_ref, val_ref, idx_ref):
    t_ = ts_ref[0]
    idx_ref[0] = (x_ref[0] + float(W) * y_ref[0]).astype(jnp.int32)
    for c in range(C):
        s = t_ - float(c)
        h1 = []
        for j in range(16):
            z = w1_ref[j] * s + b1_ref[j]
            h = jnp.maximum(z, 0.1 * z)
            h1.append(h.astype(jnp.bfloat16).astype(jnp.float32))
        acc = None
        for i in range(16):
            z = b2_ref[i]
            for j in range(16):
                z = z + w2_ref[i, j] * h1[j]
            h2 = jnp.maximum(z, 0.1 * z)
            h2 = h2.astype(jnp.bfloat16).astype(jnp.float32)
            term = w3_ref[i] * h2
            acc = term if acc is None else acc + term
        val_ref[c, 0] = t_ * (acc + b3_ref[0])


def _tc_values(xq, yq, tsq, w1, b1, w2, b2, w3, b3):
    nb = xq.shape[1] // RS
    grid = (B, nb)
    ev_spec = pl.BlockSpec((1, RS, 128), lambda b_, r: (b_, r, 0))
    smem = pl.BlockSpec(memory_space=pltpu.SMEM)
    return pl.pallas_call(
        _tc_body,
        grid=grid,
        in_specs=[smem] * 6 + [ev_spec] * 3,
        out_specs=[
            pl.BlockSpec((C, 1, RS, 128), lambda b_, r: (0, b_, r, 0)),
            ev_spec,
        ],
        out_shape=[
            jax.ShapeDtypeStruct((C, B, xq.shape[1], 128), jnp.float32),
            jax.ShapeDtypeStruct((B, xq.shape[1], 128), jnp.int32),
        ],
        compiler_params=pltpu.CompilerParams(
            dimension_semantics=("parallel", "parallel")),
    )(w1, b1, w2, b2, w3, b3, xq, yq, tsq)


def _sc_scatter(values, idx, np_):
    ev_per_tile = np_ // NS
    n_chunks = ev_per_tile // KE
    b_per_core = B // NC
    mesh = plsc.VectorSubcoreMesh(core_axis_name="c", subcore_axis_name="s")

    @functools.partial(
        pl.kernel,
        out_type=jax.ShapeDtypeStruct((B * C * G,), jnp.float32),
        mesh=mesh,
        scratch_types=[
            pltpu.VMEM((KE,), jnp.int32),
            pltpu.VMEM((KE,), jnp.float32),
            pltpu.VMEM((GS,), jnp.float32),
            pltpu.VMEM_SHARED((G,), jnp.float32),
        ],
    )
    def run(val_hbm, idx_hbm, out_hbm, idx_v, val_v, zero_v, grid):
        cid = lax.axis_index("c")
        sid = lax.axis_index("s")

        def zbody(i, _):
            zero_v[pl.ds(i * 16, 16)] = jnp.zeros((16,), jnp.float32)
            return 0

        lax.fori_loop(0, GS // 16, zbody, 0)
        for bb in range(b_per_core):
            b = cid * b_per_core + bb
            for c in range(C):
                pltpu.sync_copy(zero_v, grid.at[pl.ds(sid * GS, GS)])
                plsc.subcore_barrier()

                def chunk(k, _):
                    off = sid * ev_per_tile + k * KE
                    pltpu.sync_copy(idx_hbm.at[pl.ds(b * np_ + off, KE)], idx_v)
                    pltpu.sync_copy(
                        val_hbm.at[pl.ds((c * B + b) * np_ + off, KE)], val_v)
                    pltpu.sync_copy(val_v, grid.at[idx_v], add=True)
                    return 0

                lax.fori_loop(0, n_chunks, chunk, 0)
                plsc.subcore_barrier()
                pltpu.sync_copy(grid.at[pl.ds(sid * GS, GS)],
                                out_hbm.at[pl.ds((b * C + c) * G + sid * GS, GS)])

    return run(values, idx)


def kernel(events, W1, b1, W2, b2, W3, b3):
    n = events.shape[0]
    n_per = n // B
    quantum = NS * KE
    np_ = ((n_per + quantum - 1) // quantum) * quantum
    pad = np_ - n_per

    cols = events.reshape(B, n_per, 5)
    x = cols[:, :, 0]
    y = cols[:, :, 1]
    p = cols[:, :, 3]
    t = events[:, 2]
    bcol = events[:, 4]
    for bi in range(B):
        mask = bcol == bi
        first_idx = jnp.argmax(mask)
        last_idx = n - 1 - jnp.argmax(mask[::-1])
        tb0 = t[first_idx]
        tb1 = t[last_idx]
        dtb = tb1 - tb0
        t_norm = (t - tb0) / jnp.where(dtb == 0, 1.0, dtb) * (C - 1)
        t = jnp.where(mask & (dtb != 0), t_norm, t)
    ts = (events[:, 3] * t).reshape(B, n_per)

    def prep(col):
        return jnp.pad(col, ((0, 0), (0, pad))).reshape(B, np_ // 128, 128)

    w2q = _round_bf16(W2)
    w3q = _round_bf16(W3)
    values, idx = _tc_values(prep(x), prep(y), prep(ts),
                             W1[:, 0], b1, w2q, b2, w3q[0], b3)
    out = _sc_scatter(values.reshape(C * B * np_), idx.reshape(B * np_), np_)
    return out.reshape(B, C, H, W)
